# baseline (speedup 1.0000x reference)
import functools

import jax
import jax.numpy as jnp
from jax import lax
from jax.experimental import pallas as pl
from jax.experimental.pallas import tpu as pltpu
from jax.experimental.pallas import tpu_sc as plsc

_NC = 2
_NS = 16
_L = 16
_VOCAB = 100000
_VSHIFT = 17
_VMASK = (1 << _VSHIFT) - 1
_B_SC = 32
_BB = 8
_TT = 64


def _treemax(xs):
    while len(xs) > 1:
        nxt = [jnp.maximum(xs[i], xs[i + 1]) for i in range(0, len(xs) - 1, 2)]
        if len(xs) % 2:
            nxt.append(xs[-1])
        xs = nxt
    return xs[0]


def _treeor(xs):
    while len(xs) > 1:
        nxt = [xs[i] | xs[i + 1] for i in range(0, len(xs) - 1, 2)]
        if len(xs) % 2:
            nxt.append(xs[-1])
        xs = nxt
    return xs[0]


_GDN = lax.GatherDimensionNumbers(
    offset_dims=(), collapsed_slice_dims=(0,), start_index_map=(0,))


def _bcast(x, j):
    idx = jnp.full((_L, 1), j, jnp.int32)
    return lax.gather(x, idx, _GDN, (1,),
                      mode=lax.GatherScatterMode.PROMISE_IN_BOUNDS)


def _sc_body(q_hbm, k_hbm, v_hbm, out_hbm, tab, qv, kv, vv, pv, ov):
    B, S = q_hbm.shape
    nchunks = S // _L
    rows_per_w = B // (_NC * _NS)
    wid = lax.axis_index("s") * _NC + lax.axis_index("c")
    lane = lax.iota(jnp.int32, _L)
    zero16 = jnp.zeros((_L,), jnp.int32)
    packtag = (lane + jnp.int32(1)) << _VSHIFT
    lL = jnp.int32(_L)

    for r in range(rows_per_w):
        row = wid * jnp.int32(rows_per_w) + jnp.int32(r)
        pltpu.sync_copy(q_hbm.at[row], qv)
        pltpu.sync_copy(k_hbm.at[row], kv)
        pltpu.sync_copy(v_hbm.at[row], vv)

        def zero_body(c, carry):
            for u in range(4):
                base = c * jnp.int32(4 * _L) + jnp.int32(u * _L)
                plsc.store_scatter(tab, [qv[pl.ds(base, _L)]], zero16)
                plsc.store_scatter(tab, [kv[pl.ds(base, _L)]], zero16)
                pv[pl.ds(base, _L)] = vv[pl.ds(base, _L)] | packtag
            return carry

        lax.fori_loop(jnp.int32(0), jnp.int32(nchunks // 4), zero_body,
                      jnp.int32(0), unroll=False)

        def chunk_body(c, carry):
            base = c * lL
            qc = qv[pl.ds(base, _L)]
            kc = kv[pl.ds(base, _L)]
            vc = vv[pl.ds(base, _L)]
            pc = pv[pl.ds(base, _L)]
            tabres = plsc.load_gather(tab, [qc])
            cands = []
            dups = []
            for j in range(_L):
                kj = _bcast(kc, j)
                if j < _L - 1:
                    pj = _bcast(pc, j)
                    cands.append(
                        jnp.where((qc == kj) & (lane > j), pj, zero16))
                if j > 0:
                    dups.append((kc == kj) & (lane < j))
            best = _treemax(cands)
            dup = _treeor(dups)
            plsc.store_scatter(tab, [kc], vc, mask=jnp.logical_not(dup))
            outc = jnp.where(best > jnp.int32(0), best & jnp.int32(_VMASK),
                             tabres)
            ov[pl.ds(base, _L)] = outc
            return carry

        lax.fori_loop(jnp.int32(0), jnp.int32(nchunks), chunk_body,
                      jnp.int32(0), unroll=False)
        pltpu.sync_copy(ov, out_hbm.at[row])


def _tc_body(q_ref, k_ref, v_ref, o_ref):
    S = q_ref.shape[1]
    nt = S // _TT
    tri = (lax.broadcasted_iota(jnp.int32, (_TT, _TT), 0)
           < lax.broadcasted_iota(jnp.int32, (_TT, _TT), 1))
    for bi in range(_BB):
        qrow = q_ref[bi:bi + 1, :]
        kcol = k_ref[bi:bi + 1, :].reshape(S, 1)
        vcol = v_ref[bi:bi + 1, :].reshape(S, 1)
        for cj in range(nt):
            qc = qrow[:, cj * _TT:(cj + 1) * _TT]
            acc = jnp.zeros((1, _TT), jnp.int32)
            for ti in range(cj + 1):
                kt = kcol[ti * _TT:(ti + 1) * _TT]
                vt = vcol[ti * _TT:(ti + 1) * _TT]
                tagcol = ((ti * _TT + 1 +
                           lax.broadcasted_iota(jnp.int32, (_TT, 1), 0))
                          << _VSHIFT)
                pt = vt | tagcol
                hit = kt == qc
                if ti == cj:
                    hit = hit & tri
                part = jnp.max(jnp.where(hit, pt, 0), axis=0, keepdims=True)
                acc = jnp.maximum(acc, part)
            o_ref[bi:bi + 1, cj * _TT:(cj + 1) * _TT] = acc & _VMASK


def kernel(q, k, v):
    B, S = q.shape
    low = jnp.int64(0x3FFFFFFF)
    q32 = (q & low).astype(jnp.int32)
    k32 = (k & low).astype(jnp.int32)
    v32 = (v & low).astype(jnp.int32)

    mesh = plsc.VectorSubcoreMesh(core_axis_name="c", subcore_axis_name="s")
    sc_run = functools.partial(
        pl.kernel,
        out_type=jax.ShapeDtypeStruct((_B_SC, S), jnp.int32),
        mesh=mesh,
        scratch_types=[
            pltpu.VMEM((_VOCAB,), jnp.int32),
            pltpu.VMEM((S,), jnp.int32),
            pltpu.VMEM((S,), jnp.int32),
            pltpu.VMEM((S,), jnp.int32),
            pltpu.VMEM((S,), jnp.int32),
            pltpu.VMEM((S,), jnp.int32),
        ],
        compiler_params=pltpu.CompilerParams(needs_layout_passes=False),
    )(_sc_body)
    out_sc = sc_run(q32[:_B_SC], k32[:_B_SC], v32[:_B_SC])

    B_TC = B - _B_SC
    out_tc = pl.pallas_call(
        _tc_body,
        grid=(B_TC // _BB,),
        in_specs=[
            pl.BlockSpec((_BB, S), lambda b: (b, b * 0)),
            pl.BlockSpec((_BB, S), lambda b: (b, b * 0)),
            pl.BlockSpec((_BB, S), lambda b: (b, b * 0)),
        ],
        out_specs=pl.BlockSpec((_BB, S), lambda b: (b, b * 0)),
        out_shape=jax.ShapeDtypeStruct((B_TC, S), jnp.int32),
    )(q32[_B_SC:], k32[_B_SC:], v32[_B_SC:])

    return jnp.concatenate([out_sc, out_tc], axis=0).astype(q.dtype)


# speedup vs baseline: 1.1148x; 1.1148x over previous
import functools

import jax
import jax.numpy as jnp
from jax import lax
from jax.experimental import pallas as pl
from jax.experimental.pallas import tpu as pltpu
from jax.experimental.pallas import tpu_sc as plsc

_NC = 2
_NS = 16
_L = 16
_VOCAB = 100000
_VSHIFT = 17
_VMASK = (1 << _VSHIFT) - 1
_B_SC = 32
_BB = 8
_TT = 64


def _treemax(xs):
    while len(xs) > 1:
        nxt = [jnp.maximum(xs[i], xs[i + 1]) for i in range(0, len(xs) - 1, 2)]
        if len(xs) % 2:
            nxt.append(xs[-1])
        xs = nxt
    return xs[0]


def _treeor(xs):
    while len(xs) > 1:
        nxt = [xs[i] | xs[i + 1] for i in range(0, len(xs) - 1, 2)]
        if len(xs) % 2:
            nxt.append(xs[-1])
        xs = nxt
    return xs[0]


_GDN = lax.GatherDimensionNumbers(
    offset_dims=(), collapsed_slice_dims=(0,), start_index_map=(0,))


def _bcast(x, j):
    idx = jnp.full((_L, 1), j, jnp.int32)
    return lax.gather(x, idx, _GDN, (1,),
                      mode=lax.GatherScatterMode.PROMISE_IN_BOUNDS)


def _sc_body(q_hbm, k_hbm, v_hbm, out_hbm, tab, qv, kv, vv, pv, ov):
    B, S = q_hbm.shape
    nchunks = S // _L
    rows_per_w = B // (_NC * _NS)
    wid = lax.axis_index("s") * _NC + lax.axis_index("c")
    lane = lax.iota(jnp.int32, _L)
    zero16 = jnp.zeros((_L,), jnp.int32)
    packtag = (lane + jnp.int32(1)) << _VSHIFT
    lL = jnp.int32(_L)

    for r in range(rows_per_w):
        row = wid * jnp.int32(rows_per_w) + jnp.int32(r)
        pltpu.sync_copy(q_hbm.at[row], qv)
        pltpu.sync_copy(k_hbm.at[row], kv)
        pltpu.sync_copy(v_hbm.at[row], vv)

        def zero_body(c, carry):
            for u in range(4):
                base = c * jnp.int32(4 * _L) + jnp.int32(u * _L)
                plsc.store_scatter(tab, [qv[pl.ds(base, _L)]], zero16)
                plsc.store_scatter(tab, [kv[pl.ds(base, _L)]], zero16)
                pv[pl.ds(base, _L)] = vv[pl.ds(base, _L)] | packtag
            return carry

        lax.fori_loop(jnp.int32(0), jnp.int32(nchunks // 4), zero_body,
                      jnp.int32(0), unroll=False)

        def chunk_body(c, carry):
            base = c * lL
            qc = qv[pl.ds(base, _L)]
            kc = kv[pl.ds(base, _L)]
            vc = vv[pl.ds(base, _L)]
            pc = pv[pl.ds(base, _L)]
            tabres = plsc.load_gather(tab, [qc])
            cands = []
            dups = []
            for j in range(_L):
                kj = _bcast(kc, j)
                if j < _L - 1:
                    pj = _bcast(pc, j)
                    cands.append(
                        jnp.where((qc == kj) & (lane > j), pj, zero16))
                if j > 0:
                    dups.append((kc == kj) & (lane < j))
            best = _treemax(cands)
            dup = _treeor(dups)
            plsc.store_scatter(tab, [kc], vc, mask=jnp.logical_not(dup))
            outc = jnp.where(best > jnp.int32(0), best & jnp.int32(_VMASK),
                             tabres)
            ov[pl.ds(base, _L)] = outc
            return carry

        lax.fori_loop(jnp.int32(0), jnp.int32(nchunks), chunk_body,
                      jnp.int32(0), unroll=False)
        pltpu.sync_copy(ov, out_hbm.at[row])


_CT = 128


def _tc_body(q_ref, k_ref, v_ref, o_ref):
    S = q_ref.shape[1]
    nct = S // _CT
    for bi in range(_BB):
        qrow = q_ref[bi:bi + 1, :]
        kcol = k_ref[bi:bi + 1, :].reshape(S, 1)
        vcol = v_ref[bi:bi + 1, :].reshape(S, 1)
        for cj in range(nct):
            qc = qrow[:, cj * _CT:(cj + 1) * _CT]
            acc = jnp.zeros((1, _CT), jnp.int32)
            for ti in range((cj + 1) * _CT // _TT):
                kt = kcol[ti * _TT:(ti + 1) * _TT]
                vt = vcol[ti * _TT:(ti + 1) * _TT]
                tagcol = ((ti * _TT + 1 +
                           lax.broadcasted_iota(jnp.int32, (_TT, 1), 0))
                          << _VSHIFT)
                pt = vt | tagcol
                hit = kt == qc
                if (ti + 1) * _TT > cj * _CT:
                    tp = (ti * _TT +
                          lax.broadcasted_iota(jnp.int32, (_TT, _CT), 0))
                    t = (cj * _CT +
                         lax.broadcasted_iota(jnp.int32, (_TT, _CT), 1))
                    hit = hit & (tp < t)
                part = jnp.max(jnp.where(hit, pt, 0), axis=0, keepdims=True)
                acc = jnp.maximum(acc, part)
            o_ref[bi:bi + 1, cj * _CT:(cj + 1) * _CT] = acc & _VMASK


def kernel(q, k, v):
    B, S = q.shape
    low = jnp.int64(0x3FFFFFFF)
    q32 = (q & low).astype(jnp.int32)
    k32 = (k & low).astype(jnp.int32)
    v32 = (v & low).astype(jnp.int32)

    mesh = plsc.VectorSubcoreMesh(core_axis_name="c", subcore_axis_name="s")
    sc_run = functools.partial(
        pl.kernel,
        out_type=jax.ShapeDtypeStruct((_B_SC, S), jnp.int32),
        mesh=mesh,
        scratch_types=[
            pltpu.VMEM((_VOCAB,), jnp.int32),
            pltpu.VMEM((S,), jnp.int32),
            pltpu.VMEM((S,), jnp.int32),
            pltpu.VMEM((S,), jnp.int32),
            pltpu.VMEM((S,), jnp.int32),
            pltpu.VMEM((S,), jnp.int32),
        ],
        compiler_params=pltpu.CompilerParams(needs_layout_passes=False),
    )(_sc_body)
    out_sc = sc_run(q32[:_B_SC], k32[:_B_SC], v32[:_B_SC])

    B_TC = B - _B_SC
    out_tc = pl.pallas_call(
        _tc_body,
        grid=(B_TC // _BB,),
        in_specs=[
            pl.BlockSpec((_BB, S), lambda b: (b, b * 0)),
            pl.BlockSpec((_BB, S), lambda b: (b, b * 0)),
            pl.BlockSpec((_BB, S), lambda b: (b, b * 0)),
        ],
        out_specs=pl.BlockSpec((_BB, S), lambda b: (b, b * 0)),
        out_shape=jax.ShapeDtypeStruct((B_TC, S), jnp.int32),
    )(q32[_B_SC:], k32[_B_SC:], v32[_B_SC:])

    return jnp.concatenate([out_sc, out_tc], axis=0).astype(q.dtype)


# speedup vs baseline: 1.1213x; 1.0058x over previous
import functools

import jax
import jax.numpy as jnp
from jax import lax
from jax.experimental import pallas as pl
from jax.experimental.pallas import tpu as pltpu
from jax.experimental.pallas import tpu_sc as plsc

_NC = 2
_NS = 16
_L = 16
_VOCAB = 100000
_VSHIFT = 17
_VMASK = (1 << _VSHIFT) - 1
_B_SC = 32
_BB = 16
_TT = 64


def _treemax(xs):
    while len(xs) > 1:
        nxt = [jnp.maximum(xs[i], xs[i + 1]) for i in range(0, len(xs) - 1, 2)]
        if len(xs) % 2:
            nxt.append(xs[-1])
        xs = nxt
    return xs[0]


def _treeor(xs):
    while len(xs) > 1:
        nxt = [xs[i] | xs[i + 1] for i in range(0, len(xs) - 1, 2)]
        if len(xs) % 2:
            nxt.append(xs[-1])
        xs = nxt
    return xs[0]


_GDN = lax.GatherDimensionNumbers(
    offset_dims=(), collapsed_slice_dims=(0,), start_index_map=(0,))


def _bcast(x, j):
    idx = jnp.full((_L, 1), j, jnp.int32)
    return lax.gather(x, idx, _GDN, (1,),
                      mode=lax.GatherScatterMode.PROMISE_IN_BOUNDS)


def _sc_body(q_hbm, k_hbm, v_hbm, out_hbm, tab, qv, kv, vv, pv, ov):
    B, S = q_hbm.shape
    nchunks = S // _L
    rows_per_w = B // (_NC * _NS)
    wid = lax.axis_index("s") * _NC + lax.axis_index("c")
    lane = lax.iota(jnp.int32, _L)
    zero16 = jnp.zeros((_L,), jnp.int32)
    packtag = (lane + jnp.int32(1)) << _VSHIFT
    lL = jnp.int32(_L)

    for r in range(rows_per_w):
        row = wid * jnp.int32(rows_per_w) + jnp.int32(r)
        pltpu.sync_copy(q_hbm.at[row], qv)
        pltpu.sync_copy(k_hbm.at[row], kv)
        pltpu.sync_copy(v_hbm.at[row], vv)

        def zero_body(c, carry):
            for u in range(4):
                base = c * jnp.int32(4 * _L) + jnp.int32(u * _L)
                plsc.store_scatter(tab, [qv[pl.ds(base, _L)]], zero16)
                plsc.store_scatter(tab, [kv[pl.ds(base, _L)]], zero16)
                pv[pl.ds(base, _L)] = vv[pl.ds(base, _L)] | packtag
            return carry

        lax.fori_loop(jnp.int32(0), jnp.int32(nchunks // 4), zero_body,
                      jnp.int32(0), unroll=False)

        def chunk_body(c, carry):
            base = c * lL
            qc = qv[pl.ds(base, _L)]
            kc = kv[pl.ds(base, _L)]
            vc = vv[pl.ds(base, _L)]
            pc = pv[pl.ds(base, _L)]
            tabres = plsc.load_gather(tab, [qc])
            cands = []
            dups = []
            for j in range(_L):
                kj = _bcast(kc, j)
                if j < _L - 1:
                    pj = _bcast(pc, j)
                    cands.append(
                        jnp.where((qc == kj) & (lane > j), pj, zero16))
                if j > 0:
                    dups.append((kc == kj) & (lane < j))
            best = _treemax(cands)
            dup = _treeor(dups)
            plsc.store_scatter(tab, [kc], vc, mask=jnp.logical_not(dup))
            outc = jnp.where(best > jnp.int32(0), best & jnp.int32(_VMASK),
                             tabres)
            ov[pl.ds(base, _L)] = outc
            return carry

        lax.fori_loop(jnp.int32(0), jnp.int32(nchunks), chunk_body,
                      jnp.int32(0), unroll=False)
        pltpu.sync_copy(ov, out_hbm.at[row])


_CT = 128


def _tc_body(q_ref, k_ref, v_ref, o_ref):
    S = q_ref.shape[1]
    nct = S // _CT
    for bi in range(_BB):
        qrow = q_ref[bi:bi + 1, :]
        kcol = k_ref[bi:bi + 1, :].reshape(S, 1)
        vcol = v_ref[bi:bi + 1, :].reshape(S, 1)
        for cj in range(nct):
            qc = qrow[:, cj * _CT:(cj + 1) * _CT]
            acc = jnp.zeros((_TT, _CT), jnp.int32)
            for ti in range((cj + 1) * _CT // _TT):
                kt = kcol[ti * _TT:(ti + 1) * _TT]
                vt = vcol[ti * _TT:(ti + 1) * _TT]
                tagcol = ((ti * _TT + 1 +
                           lax.broadcasted_iota(jnp.int32, (_TT, 1), 0))
                          << _VSHIFT)
                pt = vt | tagcol
                hit = kt == qc
                if (ti + 1) * _TT > cj * _CT:
                    tp = (ti * _TT +
                          lax.broadcasted_iota(jnp.int32, (_TT, _CT), 0))
                    t = (cj * _CT +
                         lax.broadcasted_iota(jnp.int32, (_TT, _CT), 1))
                    hit = hit & (tp < t)
                acc = jnp.maximum(acc, jnp.where(hit, pt, 0))
            red = jnp.max(acc, axis=0, keepdims=True)
            o_ref[bi:bi + 1, cj * _CT:(cj + 1) * _CT] = red & _VMASK


def kernel(q, k, v):
    B, S = q.shape
    low = jnp.int64(0x3FFFFFFF)
    q32 = (q & low).astype(jnp.int32)
    k32 = (k & low).astype(jnp.int32)
    v32 = (v & low).astype(jnp.int32)

    mesh = plsc.VectorSubcoreMesh(core_axis_name="c", subcore_axis_name="s")
    sc_run = functools.partial(
        pl.kernel,
        out_type=jax.ShapeDtypeStruct((_B_SC, S), jnp.int32),
        mesh=mesh,
        scratch_types=[
            pltpu.VMEM((_VOCAB,), jnp.int32),
            pltpu.VMEM((S,), jnp.int32),
            pltpu.VMEM((S,), jnp.int32),
            pltpu.VMEM((S,), jnp.int32),
            pltpu.VMEM((S,), jnp.int32),
            pltpu.VMEM((S,), jnp.int32),
        ],
        compiler_params=pltpu.CompilerParams(needs_layout_passes=False),
    )(_sc_body)
    out_sc = sc_run(q32[:_B_SC], k32[:_B_SC], v32[:_B_SC])

    B_TC = B - _B_SC
    out_tc = pl.pallas_call(
        _tc_body,
        grid=(B_TC // _BB,),
        in_specs=[
            pl.BlockSpec((_BB, S), lambda b: (b, b * 0)),
            pl.BlockSpec((_BB, S), lambda b: (b, b * 0)),
            pl.BlockSpec((_BB, S), lambda b: (b, b * 0)),
        ],
        out_specs=pl.BlockSpec((_BB, S), lambda b: (b, b * 0)),
        out_shape=jax.ShapeDtypeStruct((B_TC, S), jnp.int32),
    )(q32[_B_SC:], k32[_B_SC:], v32[_B_SC:])

    return jnp.concatenate([out_sc, out_tc], axis=0).astype(q.dtype)
